# Initial kernel scaffold; baseline (speedup 1.0000x reference)
import jax
import jax.numpy as jnp
from jax.experimental import pallas as pl


def kernel(x, edge_index, W_pre, b_pre, W1, b1, W2, b2):
    raise NotImplementedError("write your pallas kernel here")


# speedup vs baseline: 12.9369x; 12.9369x over previous
import functools

import jax
import jax.numpy as jnp
from jax import lax
from jax.experimental import pallas as pl
from jax.experimental.pallas import tpu as pltpu
from jax.experimental.pallas import tpu_sc as plsc

N = 10000
E = 320000
F = 128
NPAD = 10240
NC, NS = 2, 16
NW = NC * NS
EPW = E // NW
CHUNK = 80
NCHUNK = EPW // CHUNK
RPT = NPAD // NS

_mesh = plsc.VectorSubcoreMesh(core_axis_name="c", subcore_axis_name="s")


def _sc_deg_body(dst_hbm, out_hbm, didx, ones_v, zrow, acc_sh):
    c = lax.axis_index("c")
    s = lax.axis_index("s")
    wid = s * NC + c
    base = wid * EPW
    for k in range(CHUNK // 16):
        ones_v[pl.ds(k * 16, 16)] = jnp.full((16,), 1.0, jnp.float32)
    for k in range(RPT // 16):
        zrow[pl.ds(k * 16, 16)] = jnp.zeros((16,), jnp.float32)
    pltpu.sync_copy(zrow, acc_sh.at[pl.ds(s * RPT, RPT)])
    plsc.subcore_barrier()

    def body(i, carry):
        pltpu.sync_copy(dst_hbm.at[pl.ds(base + i * CHUNK, CHUNK)], didx)
        pltpu.sync_copy(ones_v, acc_sh.at[didx], add=True)
        return carry

    lax.fori_loop(0, NCHUNK, body, 0)
    plsc.subcore_barrier()
    pltpu.sync_copy(acc_sh.at[pl.ds(s * RPT, RPT)], out_hbm.at[c, pl.ds(s * RPT, RPT)])


_sc_deg = pl.kernel(
    _sc_deg_body,
    out_type=jax.ShapeDtypeStruct((NC, NPAD), jnp.float32),
    mesh=_mesh,
    scratch_types=[
        pltpu.VMEM((CHUNK,), jnp.int32),
        pltpu.VMEM((CHUNK,), jnp.float32),
        pltpu.VMEM((RPT,), jnp.float32),
        pltpu.VMEM_SHARED((NPAD,), jnp.float32),
    ],
)


def _sc_agg_body(g_hbm, src_hbm, dst_hbm, out_hbm, sidx, didx, rows, zbuf, acc_sh, sem):
    c = lax.axis_index("c")
    s = lax.axis_index("s")
    wid = s * NC + c
    base = wid * EPW

    def zr(r, carry):
        for k in range(F // 16):
            zbuf[r, pl.ds(k * 16, 16)] = jnp.zeros((16,), jnp.float32)
        return carry

    lax.fori_loop(0, CHUNK, zr, 0)
    for j in range(RPT // CHUNK):
        pltpu.sync_copy(zbuf, acc_sh.at[pl.ds(s * RPT + j * CHUNK, CHUNK)])
    plsc.subcore_barrier()

    def body(i, carry):
        off = base + i * CHUNK
        pltpu.sync_copy(src_hbm.at[pl.ds(off, CHUNK)], sidx)
        pltpu.sync_copy(dst_hbm.at[pl.ds(off, CHUNK)], didx)
        pltpu.async_copy(g_hbm.at[sidx], rows, sem).wait()
        pltpu.sync_copy(rows, acc_sh.at[didx], add=True)
        return carry

    lax.fori_loop(0, NCHUNK, body, 0)
    plsc.subcore_barrier()
    pltpu.sync_copy(acc_sh.at[pl.ds(s * RPT, RPT)], out_hbm.at[c, pl.ds(s * RPT, RPT)])


_sc_agg = pl.kernel(
    _sc_agg_body,
    out_type=jax.ShapeDtypeStruct((NC, NPAD, F), jnp.float32),
    mesh=_mesh,
    scratch_types=[
        pltpu.VMEM((CHUNK,), jnp.int32),
        pltpu.VMEM((CHUNK,), jnp.int32),
        pltpu.VMEM((CHUNK, F), jnp.float32),
        pltpu.VMEM((CHUNK, F), jnp.float32),
        pltpu.VMEM_SHARED((NPAD, F), jnp.float32),
        pltpu.SemaphoreType.DMA,
    ],
)


def _tc_dinv_body(p_ref, o_ref):
    d = p_ref[0] + p_ref[1] + 1.0
    o_ref[...] = lax.rsqrt(d)


_tc_dinv = pl.pallas_call(
    _tc_dinv_body,
    out_shape=jax.ShapeDtypeStruct((NPAD // F, F), jnp.float32),
)

_R = 1000


def _tc_pre_body(x_ref, wp_ref, bp_ref, w1_ref, dv_ref, h0_ref, gs1_ref):
    h0 = jnp.dot(x_ref[...], wp_ref[...], preferred_element_type=jnp.float32)
    h0 = h0 + bp_ref[...]
    h0_ref[...] = h0
    g1 = jnp.dot(h0, w1_ref[...], preferred_element_type=jnp.float32)
    gs1_ref[...] = g1 * dv_ref[...]


_tc_pre = pl.pallas_call(
    _tc_pre_body,
    grid=(N // _R,),
    in_specs=[
        pl.BlockSpec((_R, F), lambda i: (i, 0)),
        pl.BlockSpec((F, F), lambda i: (0, 0)),
        pl.BlockSpec((1, F), lambda i: (0, 0)),
        pl.BlockSpec((F, F), lambda i: (0, 0)),
        pl.BlockSpec((_R, 1), lambda i: (i, 0)),
    ],
    out_specs=[
        pl.BlockSpec((_R, F), lambda i: (i, 0)),
        pl.BlockSpec((_R, F), lambda i: (i, 0)),
    ],
    out_shape=[
        jax.ShapeDtypeStruct((N, F), jnp.float32),
        jax.ShapeDtypeStruct((N, F), jnp.float32),
    ],
)


def _tc_mid_body(h0_ref, gs1_ref, a0_ref, a1_ref, dv_ref, b1_ref, w2_ref, u_ref, gs2_ref):
    u = h0_ref[...] + dv_ref[...] * (a0_ref[...] + a1_ref[...] + gs1_ref[...]) + b1_ref[...]
    u_ref[...] = u
    g2 = jnp.dot(u, w2_ref[...], preferred_element_type=jnp.float32)
    gs2_ref[...] = g2 * dv_ref[...]


_tc_mid = pl.pallas_call(
    _tc_mid_body,
    grid=(N // _R,),
    in_specs=[
        pl.BlockSpec((_R, F), lambda i: (i, 0)),
        pl.BlockSpec((_R, F), lambda i: (i, 0)),
        pl.BlockSpec((_R, F), lambda i: (i, 0)),
        pl.BlockSpec((_R, F), lambda i: (i, 0)),
        pl.BlockSpec((_R, 1), lambda i: (i, 0)),
        pl.BlockSpec((1, F), lambda i: (0, 0)),
        pl.BlockSpec((F, F), lambda i: (0, 0)),
    ],
    out_specs=[
        pl.BlockSpec((_R, F), lambda i: (i, 0)),
        pl.BlockSpec((_R, F), lambda i: (i, 0)),
    ],
    out_shape=[
        jax.ShapeDtypeStruct((N, F), jnp.float32),
        jax.ShapeDtypeStruct((N, F), jnp.float32),
    ],
)


def _tc_fin_body(u_ref, gs2_ref, a0_ref, a1_ref, dv_ref, b2_ref, o_ref):
    t = u_ref[...] + dv_ref[...] * (a0_ref[...] + a1_ref[...] + gs2_ref[...]) + b2_ref[...]
    o_ref[...] = jnp.where(t >= 0, t, 0.1 * t)


_tc_fin = pl.pallas_call(
    _tc_fin_body,
    grid=(N // _R,),
    in_specs=[
        pl.BlockSpec((_R, F), lambda i: (i, 0)),
        pl.BlockSpec((_R, F), lambda i: (i, 0)),
        pl.BlockSpec((_R, F), lambda i: (i, 0)),
        pl.BlockSpec((_R, F), lambda i: (i, 0)),
        pl.BlockSpec((_R, 1), lambda i: (i, 0)),
        pl.BlockSpec((1, F), lambda i: (0, 0)),
    ],
    out_specs=pl.BlockSpec((_R, F), lambda i: (i, 0)),
    out_shape=jax.ShapeDtypeStruct((N, F), jnp.float32),
)


def kernel(x, edge_index, W_pre, b_pre, W1, b1, W2, b2):
    src = edge_index[0].astype(jnp.int32)
    dst = edge_index[1].astype(jnp.int32)
    degp = _sc_deg(dst)
    dinv2d = _tc_dinv(degp.reshape(NC, NPAD // F, F))
    dinv = dinv2d.reshape(NPAD, 1)[:N]
    h0, gs1 = _tc_pre(x, W_pre, b_pre.reshape(1, F), W1, dinv)
    agg1 = _sc_agg(gs1, src, dst)
    u, gs2 = _tc_mid(h0, gs1, agg1[0, :N], agg1[1, :N], dinv, b1.reshape(1, F), W2)
    agg2 = _sc_agg(gs2, src, dst)
    return _tc_fin(u, gs2, agg2[0, :N], agg2[1, :N], dinv, b2.reshape(1, F))
